# baseline (speedup 1.0000x reference)
import jax
import jax.numpy as jnp
from jax.experimental import pallas as pl
from jax.experimental.pallas import tpu as pltpu


def _gcn_body(x_ref, adj_ref, w1_ref, b1_ref, w2_ref, b2_ref, out_ref,
              s1_ref, s2_ref):
    p = pl.program_id(0)
    i = pl.program_id(1)
    bi = adj_ref.shape[0]

    @pl.when((p == 0) & (i == 0))
    def _init():
        s1_ref[...] = jnp.dot(x_ref[...], w1_ref[...],
                              preferred_element_type=jnp.float32)

    @pl.when(p == 0)
    def _layer1():
        a = adj_ref[...].astype(jnp.bfloat16)
        s1 = s1_ref[...].astype(jnp.bfloat16)
        h = jnp.dot(a, s1, preferred_element_type=jnp.float32) + b1_ref[...]
        h = jnp.maximum(h, 0.0)
        s2_ref[pl.ds(i * bi, bi), :] = jnp.dot(
            h, w2_ref[...], preferred_element_type=jnp.float32)

    @pl.when(p == 1)
    def _layer2():
        a = adj_ref[...].astype(jnp.bfloat16)
        s2 = s2_ref[...].astype(jnp.bfloat16)
        o = jnp.dot(a, s2, preferred_element_type=jnp.float32) + b2_ref[...]
        m = jnp.max(o, axis=1, keepdims=True)
        e = o - m
        lse = jnp.log(jnp.sum(jnp.exp(e), axis=1, keepdims=True))
        out_ref[...] = e - lse


def _pick_block(n: int) -> int:
    for cand in (400, 200, 100, 80, 40, 16, 8):
        if n % cand == 0:
            return cand
    return n


def kernel(x, adj, W1, b1, W2, b2):
    n, nfeat = x.shape
    nhid = W1.shape[1]
    ncls = W2.shape[1]
    bi = _pick_block(n)
    ni = n // bi

    return pl.pallas_call(
        _gcn_body,
        grid=(2, ni),
        in_specs=[
            pl.BlockSpec((n, nfeat), lambda p, i: (0, 0)),
            pl.BlockSpec((bi, n), lambda p, i: (i, 0)),
            pl.BlockSpec((nfeat, nhid), lambda p, i: (0, 0)),
            pl.BlockSpec((1, nhid), lambda p, i: (0, 0)),
            pl.BlockSpec((nhid, ncls), lambda p, i: (0, 0)),
            pl.BlockSpec((1, ncls), lambda p, i: (0, 0)),
        ],
        out_specs=pl.BlockSpec((bi, ncls), lambda p, i: (i, 0)),
        out_shape=jax.ShapeDtypeStruct((n, ncls), jnp.float32),
        scratch_shapes=[
            pltpu.VMEM((n, nhid), jnp.float32),
            pltpu.VMEM((n, ncls), jnp.float32),
        ],
    )(x, adj, W1, b1.reshape(1, -1), W2, b2.reshape(1, -1))
